# baseline (speedup 1.0000x reference)
import functools

import jax
import jax.numpy as jnp
from jax import lax
from jax.experimental import pallas as pl
from jax.experimental.pallas import tpu as pltpu
from jax.experimental.pallas import tpu_sc as plsc

NUM_CORES = 2
NUM_SUBCORES = 16
NUM_WORKERS = NUM_CORES * NUM_SUBCORES
CHUNK = 128


@functools.cache
def _build(B, D):
    b_per_w = B // NUM_WORKERS
    n_chunks = b_per_w // CHUNK
    mesh = plsc.VectorSubcoreMesh(core_axis_name="c", subcore_axis_name="s")

    @functools.partial(
        pl.kernel,
        out_type=jax.ShapeDtypeStruct((B, D), jnp.float32),
        mesh=mesh,
        scratch_types=[
            pltpu.VMEM((n_chunks, CHUNK), jnp.int32),
            pltpu.VMEM((b_per_w, D), jnp.float32),
            pltpu.SemaphoreType.DMA,
        ],
        compiler_params=pltpu.CompilerParams(use_tc_tiling_on_sc=False),
    )
    def gather_kernel(idx_hbm, table_hbm, out_hbm, idx_v, rows_v, sem):
        wid = lax.axis_index("s") * NUM_CORES + lax.axis_index("c")
        pltpu.sync_copy(idx_hbm.at[wid], idx_v)
        copies = [
            pltpu.async_copy(
                table_hbm.at[idx_v.at[j]],
                rows_v.at[pl.ds(j * CHUNK, CHUNK)],
                sem,
            )
            for j in range(n_chunks)
        ]
        for c in copies:
            c.wait()
        pltpu.sync_copy(rows_v, out_hbm.at[pl.ds(wid * b_per_w, b_per_w)])

    return gather_kernel


@jax.jit
def kernel(x, table):
    B = x.shape[0]
    D = table.shape[1]
    idx = x.astype(jnp.int32).reshape(NUM_WORKERS, B // (NUM_WORKERS * CHUNK), CHUNK)
    return _build(B, D)(idx, table)


# speedup vs baseline: 2.3765x; 2.3765x over previous
import functools

import jax
import jax.numpy as jnp
from jax import lax
from jax.experimental import pallas as pl
from jax.experimental.pallas import tpu as pltpu
from jax.experimental.pallas import tpu_sc as plsc

NUM_CORES = 2
NUM_SUBCORES = 16
NW = NUM_CORES * NUM_SUBCORES
B = 16384
V = 1000000
D = 64
CW = 512
CSH = 9
NFULL = V // CW
TAIL_W = V - NFULL * CW
NCH = NFULL + 1
MID = B * D
MIDP = MID + 16384
IP = 1024
NP = B // IP
PSH = 10

_i32 = jnp.int32


def _iota16():
    return lax.iota(_i32, 16)


@functools.cache
def _build_phase_a():
    mesh = plsc.VectorSubcoreMesh(core_axis_name="c", subcore_axis_name="s")

    @functools.partial(
        pl.kernel,
        out_type=(
            jax.ShapeDtypeStruct((MID,), jnp.float32),
            jax.ShapeDtypeStruct((MID,), jnp.float32),
        ),
        mesh=mesh,
        scratch_types=[
            pltpu.VMEM((IP,), _i32),
            pltpu.VMEM((IP,), _i32),
            pltpu.VMEM((D, TAIL_W), jnp.float32),
            pltpu.VMEM((D, CW), jnp.float32),
            pltpu.VMEM((B + 16,), jnp.float32),
            pltpu.VMEM((1024,), jnp.float32),
            pltpu.VMEM((1024,), jnp.float32),
            pltpu.VMEM((1024,), _i32),
            pltpu.VMEM((1024,), _i32),
            pltpu.VMEM((64,), _i32),
            pltpu.VMEM((64,), _i32),
            pltpu.SMEM((128,), _i32),
            pltpu.VMEM_SHARED((MIDP,), jnp.float32),
            pltpu.SemaphoreType.DMA,
            pltpu.SemaphoreType.DMA,
            pltpu.SemaphoreType.DMA,
            pltpu.SemaphoreType.DMA,
            pltpu.SemaphoreType.DMA,
            pltpu.SemaphoreType.DMA,
        ],
        compiler_params=pltpu.CompilerParams(needs_layout_passes=False),
    )
    def phase_a(x_hbm, tableT_hbm, tail_hbm, mid0_hbm, mid1_hbm,
                ib0, ib1, tailbuf, win, wl, sdat0, sdat1, sidx0, sidx1,
                cnt, ends, bnd_sm, acc_sh, sem, sem_t, sem_z, sem_i,
                ssem0, ssem1):
        cid = lax.axis_index("c")
        sid = lax.axis_index("s")
        wid = sid * NUM_CORES + cid
        lo = (wid * NCH) >> 5
        hi = ((wid + 1) * NCH) >> 5
        nloc = hi - lo
        iota = _iota16()
        zeros16f = lax.full((16,), 0.0, jnp.float32)
        zeros16 = lax.full((16,), 0, _i32)

        tail_cp = pltpu.async_copy(tail_hbm, tailbuf, sem_t)

        def zb_body(i, _):
            wl[pl.ds(i * 16, 16)] = zeros16f
            return 0

        lax.fori_loop(0, B // 16, zb_body, 0)
        for g in range(4):
            cnt[pl.ds(g * 16, 16)] = zeros16
        stripe = sid * (MID // 16)
        zcopies = [
            pltpu.async_copy(
                wl.at[pl.ds(0, B)], acc_sh.at[pl.ds(stripe + k * B, B)], sem_z
            )
            for k in range(MID // 16 // B)
        ]

        def stream_idx(process_group):
            bufs = (ib0, ib1)
            cur = pltpu.async_copy(x_hbm.at[pl.ds(0, IP)], ib0, sem_i)
            for p in range(NP):
                cur.wait()
                if p + 1 < NP:
                    nxt = pltpu.async_copy(
                        x_hbm.at[pl.ds((p + 1) * IP, IP)], bufs[(p + 1) & 1],
                        sem_i,
                    )
                buf = bufs[p & 1]

                def g_body(g, _, _p=p, _buf=buf):
                    rv = _buf[pl.ds(g * 16, 16)]
                    process_group(_p * IP + g * 16, g, rv)
                    return 0

                lax.fori_loop(0, IP // 16, g_body, 0)
                if p + 1 < NP:
                    cur = nxt

        def hist_group(base, g, rv):
            del base, g
            ch = lax.shift_right_logical(rv, CSH)
            mine = (ch >= lo) & (ch < hi)
            lcl = ch - lo
            dup, last = plsc.scan_count(lcl, mine)
            plsc.addupdate_scatter(cnt, [lcl], dup, mask=mine & last)

        stream_idx(hist_group)

        carry = 0
        for g in range(4):
            cg = cnt[pl.ds(g * 16, 16)]
            inc = plsc.cumsum(cg) + carry
            exc = inc - cg
            ends[pl.ds(g * 16, 16)] = exc
            for u in range(16):
                bnd_sm[g * 16 + u] = exc[u]
            carry = inc[15]

        def place_group(base, g, rv):
            ch = lax.shift_right_logical(rv, CSH)
            mine = (ch >= lo) & (ch < hi)
            lcl = ch - lo
            dup, last = plsc.scan_count(lcl, mine)
            bs = plsc.load_gather(ends, [lcl], mask=mine)
            slot = bs + dup - 1
            lc = rv & (CW - 1)
            pos = base + iota
            packed = plsc.bitcast(lc | (pos << PSH), jnp.float32)
            plsc.store_scatter(wl, [slot], packed, mask=mine)
            plsc.addupdate_scatter(ends, [lcl], dup, mask=mine & last)

        stream_idx(place_group)

        for g in range(4):
            ev = ends[pl.ds(g * 16, 16)]
            for u in range(16):
                bnd_sm[64 + g * 16 + u] = ev[u]

        for zc in zcopies:
            zc.wait()
        tail_cp.wait()
        plsc.subcore_barrier()

        dump = MID + sid * 1024
        for si, sd, ss in ((sidx0, sdat0, ssem0), (sidx1, sdat1, ssem1)):
            def prime_body(q, _, _si=si):
                _si[pl.ds(q * 16, 16)] = dump + q * 16 + iota
                return 0

            lax.fori_loop(0, 64, prime_body, 0)
            pltpu.async_copy(sd, acc_sh.at[si], ss)

        def chunk_body(m, _):
            ch = lo + m
            b0 = bnd_sm[m]
            b1 = bnd_sm[64 + m]
            is_tail_chunk = ch == NCH - 1

            @pl.when(jnp.logical_not(is_tail_chunk))
            def _():
                pltpu.async_copy(
                    tableT_hbm.at[:, pl.ds(pl.multiple_of(ch * CW, CW), CW)],
                    win, sem,
                ).wait()

            nb = lax.shift_right_logical(b1 - b0 + 15, 4)

            def batch_body(bb, _):
                base = b0 + bb * 16
                wv = plsc.bitcast(wl[pl.ds(base, 16)], _i32)
                rem = b1 - base
                mask = iota < rem
                lc = wv & ((1 << PSH) - 1)
                pos = lax.shift_right_logical(wv, PSH)
                pos_eff = jnp.where(mask, pos, B + sid * 16 + iota)
                par = bb & 1

                def do_par(sd, si, ss, src):
                    pltpu.make_async_copy(sd, acc_sh.at[si], ss).wait()

                    def feat_body(cg, _):
                        for cu in range(4):
                            c = cg * 4 + cu
                            cvec = lax.full((16,), 0, _i32) + c
                            v = plsc.load_gather(src, [cvec, lc], mask=mask)
                            sd[pl.ds(c * 16, 16)] = v
                            si[pl.ds(c * 16, 16)] = pos_eff * D + c
                        return 0

                    lax.fori_loop(0, 16, feat_body, 0)
                    pltpu.async_copy(sd, acc_sh.at[si], ss)

                @pl.when(jnp.logical_not(is_tail_chunk))
                def _():
                    @pl.when(par == 0)
                    def _():
                        do_par(sdat0, sidx0, ssem0, win)

                    @pl.when(par == 1)
                    def _():
                        do_par(sdat1, sidx1, ssem1, win)

                @pl.when(is_tail_chunk)
                def _():
                    @pl.when(par == 0)
                    def _():
                        do_par(sdat0, sidx0, ssem0, tailbuf)

                    @pl.when(par == 1)
                    def _():
                        do_par(sdat1, sidx1, ssem1, tailbuf)

                return 0

            lax.fori_loop(0, nb, batch_body, 0)
            return 0

        lax.fori_loop(0, nloc, chunk_body, 0)

        pltpu.make_async_copy(sdat0, acc_sh.at[sidx0], ssem0).wait()
        pltpu.make_async_copy(sdat1, acc_sh.at[sidx1], ssem1).wait()
        plsc.subcore_barrier()

        def flush(dst_hbm):
            for k in range(MID // 16 // B):
                off = stripe + k * B
                pltpu.async_copy(acc_sh.at[pl.ds(off, B)], wl.at[pl.ds(0, B)], sem).wait()
                pltpu.async_copy(wl.at[pl.ds(0, B)], dst_hbm.at[pl.ds(off, B)], sem).wait()

        @pl.when(cid == 0)
        def _():
            flush(mid0_hbm)

        @pl.when(cid == 1)
        def _():
            flush(mid1_hbm)

    return phase_a


@functools.cache
def _build_phase_b():
    mesh = plsc.VectorSubcoreMesh(core_axis_name="c", subcore_axis_name="s")
    bpw = B // NW

    @functools.partial(
        pl.kernel,
        out_type=jax.ShapeDtypeStruct((D, B), jnp.float32),
        mesh=mesh,
        scratch_types=[
            pltpu.VMEM((bpw * D,), jnp.float32),
            pltpu.VMEM((bpw * D,), jnp.float32),
            pltpu.VMEM((D, bpw), jnp.float32),
            pltpu.SemaphoreType.DMA,
        ],
        compiler_params=pltpu.CompilerParams(needs_layout_passes=False),
    )
    def phase_b(mid0_hbm, mid1_hbm, outT_hbm, m0, m1, tbuf, sem):
        cid = lax.axis_index("c")
        sid = lax.axis_index("s")
        wid = sid * NUM_CORES + cid
        base = wid * bpw * D
        cp0 = pltpu.async_copy(mid0_hbm.at[pl.ds(base, bpw * D)], m0, sem)
        cp1 = pltpu.async_copy(mid1_hbm.at[pl.ds(base, bpw * D)], m1, sem)
        cp0.wait()
        cp1.wait()
        iota = _iota16()

        def col_body(q, _):
            rowsel = q * 16 + iota
            for c in range(D):
                idx = rowsel * D + c
                v = plsc.load_gather(m0, [idx]) + plsc.load_gather(m1, [idx])
                tbuf[c, pl.ds(q * 16, 16)] = v
            return 0

        lax.fori_loop(0, bpw // 16, col_body, 0)
        pltpu.sync_copy(
            tbuf, outT_hbm.at[:, pl.ds(pl.multiple_of(wid * bpw, bpw), bpw)]
        )

    return phase_b


@jax.jit
def kernel(x, table):
    tT = table.T
    tail = lax.slice(tT, (0, NFULL * CW), (D, V))
    mid0, mid1 = _build_phase_a()(x.astype(_i32), tT, tail)
    outT = _build_phase_b()(mid0, mid1)
    return outT.T


# speedup vs baseline: 2.6327x; 1.1078x over previous
import functools

import jax
import jax.numpy as jnp
from jax import lax
from jax.experimental import pallas as pl
from jax.experimental.pallas import tpu as pltpu
from jax.experimental.pallas import tpu_sc as plsc

NUM_CORES = 2
NUM_SUBCORES = 16
NW = NUM_CORES * NUM_SUBCORES
B = 16384
V = 1000000
D = 64
CW = 256
CSH = 8
NFULL = V // CW
TAIL_W = V - NFULL * CW
NCH = NFULL + 1
MID = B * D
MIDP = MID + 16384
IP = 4096
NP = B // IP
PSH = 10

_i32 = jnp.int32


def _iota16():
    return lax.iota(_i32, 16)


@functools.cache
def _build_phase_a():
    mesh = plsc.VectorSubcoreMesh(core_axis_name="c", subcore_axis_name="s")

    @functools.partial(
        pl.kernel,
        out_type=(
            jax.ShapeDtypeStruct((MID,), jnp.float32),
            jax.ShapeDtypeStruct((MID,), jnp.float32),
        ),
        mesh=mesh,
        scratch_types=[
            pltpu.VMEM((IP,), _i32),
            pltpu.VMEM((8, 512), jnp.float32),
            pltpu.VMEM((D, CW), jnp.float32),
            pltpu.VMEM((D, CW), jnp.float32),
            pltpu.VMEM((B + 16,), jnp.float32),
            pltpu.VMEM((1024,), jnp.float32),
            pltpu.VMEM((1024,), jnp.float32),
            pltpu.VMEM((1024,), _i32),
            pltpu.VMEM((1024,), _i32),
            pltpu.VMEM((128,), _i32),
            pltpu.VMEM((128,), _i32),
            pltpu.SMEM((256,), _i32),
            pltpu.VMEM_SHARED((MIDP,), jnp.float32),
            pltpu.SemaphoreType.DMA,
            pltpu.SemaphoreType.DMA,
            pltpu.SemaphoreType.DMA,
            pltpu.SemaphoreType.DMA,
            pltpu.SemaphoreType.DMA,
            pltpu.SemaphoreType.DMA,
        ],
        compiler_params=pltpu.CompilerParams(needs_layout_passes=False),
    )
    def phase_a(x_hbm, tableT_hbm, tail_hbm, mid0_hbm, mid1_hbm,
                ib0, tailbuf, win0, win1, wl, sdat0, sdat1, sidx0, sidx1,
                cnt, ends, bnd_sm, acc_sh, sem, sem_t, sem_z, sem_i,
                ssem0, ssem1):
        cid = lax.axis_index("c")
        sid = lax.axis_index("s")
        wid = sid * NUM_CORES + cid
        lo = (wid * NCH) >> 5
        hi = ((wid + 1) * NCH) >> 5
        nloc = hi - lo
        iota = _iota16()
        zeros16f = lax.full((16,), 0.0, jnp.float32)
        zeros16 = lax.full((16,), 0, _i32)

        tail_cp = pltpu.async_copy(tail_hbm, tailbuf, sem_t)

        def zb_body(i, _):
            wl[pl.ds(i * 16, 16)] = zeros16f
            return 0

        lax.fori_loop(0, B // 16, zb_body, 0)
        for g in range(8):
            cnt[pl.ds(g * 16, 16)] = zeros16
        stripe = sid * (MID // 16)
        zcopies = [
            pltpu.async_copy(
                wl.at[pl.ds(0, B)], acc_sh.at[pl.ds(stripe + k * B, B)], sem_z
            )
            for k in range(MID // 16 // B)
        ]

        def stream_idx(process_group):
            for p in range(NP):
                pltpu.async_copy(
                    x_hbm.at[pl.ds(p * IP, IP)], ib0, sem_i
                ).wait()

                def g_body(g, _, _p=p):
                    rv = ib0[pl.ds(g * 16, 16)]
                    process_group(_p * IP + g * 16, g, rv)
                    return 0

                lax.fori_loop(0, IP // 16, g_body, 0)

        def hist_group(base, g, rv):
            del base, g
            ch = lax.shift_right_logical(rv, CSH)
            mine = (ch >= lo) & (ch < hi)
            lcl = ch - lo
            dup, last = plsc.scan_count(lcl, mine)
            plsc.addupdate_scatter(cnt, [lcl], dup, mask=mine & last)

        stream_idx(hist_group)

        carry = 0
        for g in range(8):
            cg = cnt[pl.ds(g * 16, 16)]
            inc = plsc.cumsum(cg) + carry
            exc = inc - cg
            ends[pl.ds(g * 16, 16)] = exc
            for u in range(16):
                bnd_sm[g * 16 + u] = exc[u]
            carry = inc[15]

        def place_group(base, g, rv):
            ch = lax.shift_right_logical(rv, CSH)
            mine = (ch >= lo) & (ch < hi)
            lcl = ch - lo
            dup, last = plsc.scan_count(lcl, mine)
            bs = plsc.load_gather(ends, [lcl], mask=mine)
            slot = bs + dup - 1
            lc = rv & (CW - 1)
            pos = base + iota
            packed = plsc.bitcast(lc | (pos << PSH), jnp.float32)
            plsc.store_scatter(wl, [slot], packed, mask=mine)
            plsc.addupdate_scatter(ends, [lcl], dup, mask=mine & last)

        stream_idx(place_group)

        for g in range(8):
            ev = ends[pl.ds(g * 16, 16)]
            for u in range(16):
                bnd_sm[128 + g * 16 + u] = ev[u]

        for zc in zcopies:
            zc.wait()
        tail_cp.wait()
        plsc.subcore_barrier()

        dump = MID + sid * 1024
        for si, sd, ss in ((sidx0, sdat0, ssem0), (sidx1, sdat1, ssem1)):
            def prime_body(q, _, _si=si):
                _si[pl.ds(q * 16, 16)] = dump + q * 16 + iota
                return 0

            lax.fori_loop(0, 64, prime_body, 0)
            pltpu.async_copy(sd, acc_sh.at[si], ss)

        def fire_chunk(ch, wref):
            @pl.when(ch < NCH - 1)
            def _():
                pltpu.async_copy(
                    tableT_hbm.at[:, pl.ds(pl.multiple_of(ch * CW, CW), CW)],
                    wref, sem,
                )

        fire_chunk(lo, win0)

        def chunk_body(m, _):
            ch = lo + m
            b0 = bnd_sm[m]
            b1 = bnd_sm[128 + m]
            is_tail_chunk = ch == NCH - 1
            par2 = m & 1

            @pl.when((m + 1 < nloc) & (par2 == 0))
            def _():
                fire_chunk(ch + 1, win1)

            @pl.when((m + 1 < nloc) & (par2 == 1))
            def _():
                fire_chunk(ch + 1, win0)

            @pl.when(jnp.logical_not(is_tail_chunk))
            def _():
                pltpu.make_async_copy(
                    tableT_hbm.at[:, pl.ds(0, CW)], win0, sem
                ).wait()

            nb = lax.shift_right_logical(b1 - b0 + 15, 4)

            def process(src, src_is_tail):
                def batch_body(bb, _):
                    base = b0 + bb * 16
                    wv = plsc.bitcast(wl[pl.ds(base, 16)], _i32)
                    rem = b1 - base
                    mask = iota < rem
                    lc = wv & ((1 << PSH) - 1)
                    pos = lax.shift_right_logical(wv, PSH)
                    pos_eff = jnp.where(mask, pos, B + sid * 16 + iota)
                    par = bb & 1

                    def do_par(sd, si, ss):
                        pltpu.make_async_copy(sd, acc_sh.at[si], ss).wait()

                        def feat_body(cg, _):
                            for cu in range(4):
                                c = cg * 4 + cu
                                cvec = lax.full((16,), 0, _i32) + c
                                if src_is_tail:
                                    flat = lc + c * TAIL_W
                                    v = plsc.load_gather(
                                        src,
                                        [lax.shift_right_logical(flat, 9),
                                         flat & 511],
                                        mask=mask,
                                    )
                                else:
                                    v = plsc.load_gather(
                                        src, [cvec, lc], mask=mask
                                    )
                                sd[pl.ds(c * 16, 16)] = v
                                si[pl.ds(c * 16, 16)] = pos_eff * D + c
                            return 0

                        lax.fori_loop(0, 16, feat_body, 0)
                        pltpu.async_copy(sd, acc_sh.at[si], ss)

                    @pl.when(par == 0)
                    def _():
                        do_par(sdat0, sidx0, ssem0)

                    @pl.when(par == 1)
                    def _():
                        do_par(sdat1, sidx1, ssem1)

                    return 0

                lax.fori_loop(0, nb, batch_body, 0)

            tl = is_tail_chunk

            @pl.when(jnp.logical_not(tl) & (par2 == 0))
            def _():
                process(win0, False)

            @pl.when(jnp.logical_not(tl) & (par2 == 1))
            def _():
                process(win1, False)

            @pl.when(tl)
            def _():
                process(tailbuf, True)

            return 0

        lax.fori_loop(0, nloc, chunk_body, 0)

        pltpu.make_async_copy(sdat0, acc_sh.at[sidx0], ssem0).wait()
        pltpu.make_async_copy(sdat1, acc_sh.at[sidx1], ssem1).wait()
        plsc.subcore_barrier()

        def flush(dst_hbm):
            for k in range(MID // 16 // B):
                off = stripe + k * B
                pltpu.async_copy(acc_sh.at[pl.ds(off, B)], wl.at[pl.ds(0, B)], sem).wait()
                pltpu.async_copy(wl.at[pl.ds(0, B)], dst_hbm.at[pl.ds(off, B)], sem).wait()

        @pl.when(cid == 0)
        def _():
            flush(mid0_hbm)

        @pl.when(cid == 1)
        def _():
            flush(mid1_hbm)

    return phase_a


@functools.cache
def _build_phase_b():
    mesh = plsc.VectorSubcoreMesh(core_axis_name="c", subcore_axis_name="s")
    bpw = B // NW

    @functools.partial(
        pl.kernel,
        out_type=jax.ShapeDtypeStruct((D, B), jnp.float32),
        mesh=mesh,
        scratch_types=[
            pltpu.VMEM((bpw * D,), jnp.float32),
            pltpu.VMEM((bpw * D,), jnp.float32),
            pltpu.VMEM((D, bpw), jnp.float32),
            pltpu.SemaphoreType.DMA,
        ],
        compiler_params=pltpu.CompilerParams(needs_layout_passes=False),
    )
    def phase_b(mid0_hbm, mid1_hbm, outT_hbm, m0, m1, tbuf, sem):
        cid = lax.axis_index("c")
        sid = lax.axis_index("s")
        wid = sid * NUM_CORES + cid
        base = wid * bpw * D
        cp0 = pltpu.async_copy(mid0_hbm.at[pl.ds(base, bpw * D)], m0, sem)
        cp1 = pltpu.async_copy(mid1_hbm.at[pl.ds(base, bpw * D)], m1, sem)
        cp0.wait()
        cp1.wait()
        iota = _iota16()

        def col_body(q, _):
            rowsel = q * 16 + iota
            for c in range(D):
                idx = rowsel * D + c
                v = plsc.load_gather(m0, [idx]) + plsc.load_gather(m1, [idx])
                tbuf[c, pl.ds(q * 16, 16)] = v
            return 0

        lax.fori_loop(0, bpw // 16, col_body, 0)
        pltpu.sync_copy(
            tbuf, outT_hbm.at[:, pl.ds(pl.multiple_of(wid * bpw, bpw), bpw)]
        )

    return phase_b


@jax.jit
def kernel(x, table):
    tT = table.T
    tail = lax.slice(tT, (0, NFULL * CW), (D, V)).reshape(8, 512)
    mid0, mid1 = _build_phase_a()(x.astype(_i32), tT, tail)
    outT = _build_phase_b()(mid0, mid1)
    return outT.T
